# Initial kernel scaffold; baseline (speedup 1.0000x reference)
import jax
import jax.numpy as jnp
from jax.experimental import pallas as pl


def kernel(text, embedding_weight):
    raise NotImplementedError("write your pallas kernel here")


# speedup vs baseline: 1.8338x; 1.8338x over previous
import functools

import jax
import jax.numpy as jnp
from jax import lax
from jax.experimental import pallas as pl
from jax.experimental.pallas import tpu as pltpu
from jax.experimental.pallas import tpu_sc as plsc

SEQ = 200
BATCH = 4096
D = 32
NC = 2
NS = 16
NW = NC * NS
BPW = BATCH // NW
S = 8
NCHUNK = SEQ // S
LANES = 16


def _emb_avg(text_hbm, table_hbm, out_hbm, idx_v, rows_v, dst_idx, acc_v,
             acc_sh, sem):
    cid = lax.axis_index("c")
    sid = lax.axis_index("s")
    wid = cid * NS + sid
    base = wid * BPW
    sh_base = sid * BPW

    zero16 = jnp.zeros((LANES,), jnp.float32)
    iota16 = lax.iota(jnp.int32, LANES)

    def fill_dst(i, _):
        def inner(j, _):
            dst_idx[pl.ds(i * BPW + j * LANES, LANES)] = (
                sh_base + j * LANES + iota16)
            return _
        return lax.fori_loop(0, BPW // LANES, inner, _)
    lax.fori_loop(0, S, fill_dst, None)

    def zbody(b, _):
        acc_v[b, pl.ds(0, LANES)] = zero16
        acc_v[b, pl.ds(LANES, LANES)] = zero16
        return _
    lax.fori_loop(0, BPW, zbody, None)
    pltpu.sync_copy(acc_v, acc_sh.at[pl.ds(sh_base, BPW)])

    def chunk_body(c, _):
        pltpu.sync_copy(
            text_hbm.at[pl.ds(c * S, S), pl.ds(base, BPW)], idx_v)
        copies = [
            pltpu.async_copy(
                table_hbm.at[idx_v.at[s]],
                rows_v.at[pl.ds(s * BPW, BPW)],
                sem,
            )
            for s in range(S)
        ]
        for cp in copies:
            cp.wait()
        pltpu.sync_copy(rows_v, acc_sh.at[dst_idx], add=True)
        return _

    lax.fori_loop(0, NCHUNK, chunk_body, None)

    pltpu.sync_copy(acc_sh.at[pl.ds(sh_base, BPW)], acc_v)
    inv = jnp.float32(1.0 / SEQ)

    def scale_body(b, _):
        acc_v[b, pl.ds(0, LANES)] = acc_v[b, pl.ds(0, LANES)] * inv
        acc_v[b, pl.ds(LANES, LANES)] = acc_v[b, pl.ds(LANES, LANES)] * inv
        return _
    lax.fori_loop(0, BPW, scale_body, None)

    pltpu.sync_copy(acc_v, out_hbm.at[pl.ds(base, BPW)])


def kernel(text, embedding_weight):
    text = text.astype(jnp.int32)
    mesh = plsc.VectorSubcoreMesh(core_axis_name="c", subcore_axis_name="s")
    f = functools.partial(
        pl.kernel,
        mesh=mesh,
        compiler_params=pltpu.CompilerParams(use_tc_tiling_on_sc=False),
        out_type=jax.ShapeDtypeStruct((BATCH, D), jnp.float32),
        scratch_types=[
            pltpu.VMEM((S, BPW), jnp.int32),
            pltpu.VMEM((S * BPW, D), jnp.float32),
            pltpu.VMEM((S * BPW,), jnp.int32),
            pltpu.VMEM((BPW, D), jnp.float32),
            pltpu.VMEM_SHARED((NS * BPW, D), jnp.float32),
            pltpu.SemaphoreType.DMA,
        ],
    )(_emb_avg)
    return f(text, embedding_weight)


# speedup vs baseline: 1.9565x; 1.0669x over previous
import functools

import jax
import jax.numpy as jnp
from jax import lax
from jax.experimental import pallas as pl
from jax.experimental.pallas import tpu as pltpu
from jax.experimental.pallas import tpu_sc as plsc

VOCAB = 1000000
SEQ = 200
BATCH = 4096
D = 32
NC = 2
NS = 16
NW = NC * NS
BPW = BATCH // NW
S = 10
NCHUNK = SEQ // S
LANES = 16


def _emb_avg(text_hbm, table_hbm, out_hbm, idx_v, rows_v, dst_idx, acc_v,
             acc_sh, sem0, sem1):
    cid = lax.axis_index("c")
    sid = lax.axis_index("s")
    wid = cid * NS + sid
    base = wid * BPW
    sh_base = sid * BPW
    sems = (sem0, sem1)

    zero16 = jnp.zeros((LANES,), jnp.float32)
    iota16 = lax.iota(jnp.int32, LANES)

    def fill_dst(i, _):
        def inner(j, _):
            dst_idx[pl.ds(i * BPW + j * LANES, LANES)] = (
                sh_base + j * LANES + iota16)
            return _
        return lax.fori_loop(0, BPW // LANES, inner, _)
    lax.fori_loop(0, S, fill_dst, None)

    def zbody(b, _):
        acc_v[b, pl.ds(0, LANES)] = zero16
        acc_v[b, pl.ds(LANES, LANES)] = zero16
        return _
    lax.fori_loop(0, BPW, zbody, None)
    pltpu.sync_copy(acc_v, acc_sh.at[pl.ds(sh_base, BPW)])

    def fire(c, k):
        pltpu.sync_copy(
            text_hbm.at[pl.ds(c * S, S), pl.ds(base, BPW)], idx_v.at[k])
        for s in range(S):
            pltpu.async_copy(
                table_hbm.at[idx_v.at[k, s]],
                rows_v.at[k, pl.ds(s * BPW, BPW)],
                sems[k],
            )

    def drain_and_accumulate(k):
        pltpu.make_async_copy(
            table_hbm.at[pl.ds(0, S * BPW)], rows_v.at[k], sems[k]).wait()
        pltpu.sync_copy(rows_v.at[k], acc_sh.at[dst_idx], add=True)

    fire(0, 0)

    def pair_body(i, _):
        c0 = 2 * i

        @pl.when(c0 + 1 < NCHUNK)
        def _fire1():
            fire(c0 + 1, 1)

        drain_and_accumulate(0)

        @pl.when(c0 + 2 < NCHUNK)
        def _fire0():
            fire(c0 + 2, 0)

        @pl.when(c0 + 1 < NCHUNK)
        def _drain1():
            drain_and_accumulate(1)
        return _

    lax.fori_loop(0, (NCHUNK + 1) // 2, pair_body, None)

    pltpu.sync_copy(acc_sh.at[pl.ds(sh_base, BPW)], acc_v)
    inv = jnp.float32(1.0 / SEQ)

    def scale_body(b, _):
        acc_v[b, pl.ds(0, LANES)] = acc_v[b, pl.ds(0, LANES)] * inv
        acc_v[b, pl.ds(LANES, LANES)] = acc_v[b, pl.ds(LANES, LANES)] * inv
        return _
    lax.fori_loop(0, BPW, scale_body, None)

    pltpu.sync_copy(acc_v, out_hbm.at[pl.ds(base, BPW)])


def kernel(text, embedding_weight):
    text = text.astype(jnp.int32)
    packed = lax.optimization_barrier(jnp.reshape(embedding_weight,
                                                  (VOCAB // 4, 4 * D)))
    embedding_weight = jnp.reshape(packed, (VOCAB, D))
    mesh = plsc.VectorSubcoreMesh(core_axis_name="c", subcore_axis_name="s")
    f = functools.partial(
        pl.kernel,
        mesh=mesh,
        compiler_params=pltpu.CompilerParams(use_tc_tiling_on_sc=False),
        out_type=jax.ShapeDtypeStruct((BATCH, D), jnp.float32),
        scratch_types=[
            pltpu.VMEM((2, S, BPW), jnp.int32),
            pltpu.VMEM((2, S * BPW, D), jnp.float32),
            pltpu.VMEM((S * BPW,), jnp.int32),
            pltpu.VMEM((BPW, D), jnp.float32),
            pltpu.VMEM_SHARED((NS * BPW, D), jnp.float32),
            pltpu.SemaphoreType.DMA,
            pltpu.SemaphoreType.DMA,
        ],
    )(_emb_avg)
    return f(text, embedding_weight)
